# Initial kernel scaffold; baseline (speedup 1.0000x reference)
import jax
import jax.numpy as jnp
from jax.experimental import pallas as pl


def kernel(feat, edge_index, edge_weight, W_neigh, W_self):
    raise NotImplementedError("write your pallas kernel here")


# speedup vs baseline: 5.3838x; 5.3838x over previous
import functools

import jax
import jax.numpy as jnp
from jax import lax
from jax.experimental import pallas as pl
from jax.experimental.pallas import tpu as pltpu
from jax.experimental.pallas import tpu_sc as plsc

NC = 2
NS = 16
NW = NC * NS
LANES = 16
CHUNK = 128


def _sc_aggregate(feat, src3, dst3, w3, n_acc, nch):
    d = feat.shape[1]
    rows_per_tile = n_acc // NS
    mesh = plsc.VectorSubcoreMesh(core_axis_name="c", subcore_axis_name="s")

    @functools.partial(
        pl.kernel,
        out_type=(
            jax.ShapeDtypeStruct((NC, n_acc, d), jnp.float32),
            jax.ShapeDtypeStruct((NC * n_acc,), jnp.float32),
        ),
        mesh=mesh,
        scratch_types=[
            pltpu.VMEM((nch, CHUNK), jnp.int32),
            pltpu.VMEM((nch, CHUNK), jnp.int32),
            pltpu.VMEM((nch, CHUNK), jnp.float32),
            pltpu.VMEM((CHUNK, d), jnp.float32),
            pltpu.VMEM((CHUNK,), jnp.float32),
            pltpu.VMEM((n_acc // NS,), jnp.float32),
            pltpu.VMEM_SHARED((n_acc, d), jnp.float32),
            pltpu.VMEM_SHARED((n_acc,), jnp.float32),
        ],
    )
    def agg_kernel(feat_hbm, src_hbm, dst_hbm, w_hbm, agg_hbm, deg_hbm,
                   srcv, dstv, wv, rbuf, ones, degv, acc_sh, deg_sh):
        c = lax.axis_index("c")
        s = lax.axis_index("s")
        wid = c * NS + s

        zeros16 = jnp.zeros((LANES,), jnp.float32)
        ones16 = jnp.ones((LANES,), jnp.float32)

        @pl.loop(0, CHUNK)
        def _(r):
            for j in range(d // LANES):
                rbuf[r, pl.ds(j * LANES, LANES)] = zeros16

        @pl.loop(0, CHUNK // LANES)
        def _(i):
            ones[pl.ds(i * LANES, LANES)] = ones16

        base = s * rows_per_tile
        off = 0
        while off < rows_per_tile:
            nr = min(CHUNK, rows_per_tile - off)
            pltpu.sync_copy(rbuf.at[pl.ds(0, nr)],
                            acc_sh.at[pl.ds(base + off, nr)])
            pltpu.sync_copy(rbuf.at[0, pl.ds(0, nr)],
                            deg_sh.at[pl.ds(base + off, nr)])
            off += nr
        plsc.subcore_barrier()

        pltpu.sync_copy(src_hbm.at[wid], srcv)
        pltpu.sync_copy(dst_hbm.at[wid], dstv)
        pltpu.sync_copy(w_hbm.at[wid], wv)

        @pl.loop(0, nch)
        def _(g):
            pltpu.sync_copy(feat_hbm.at[srcv.at[g]], rbuf)

            @pl.loop(0, CHUNK // LANES)
            def _(e16):
                w16 = wv[g, pl.ds(e16 * LANES, LANES)]
                for k in range(LANES):
                    wvec = jnp.full((LANES,), w16[k], jnp.float32)
                    row = e16 * LANES + k
                    for j in range(d // LANES):
                        sl = pl.ds(j * LANES, LANES)
                        rbuf[row, sl] = rbuf[row, sl] * wvec

            pltpu.sync_copy(rbuf, acc_sh.at[dstv.at[g]], add=True)
            pltpu.sync_copy(ones, deg_sh.at[dstv.at[g]], add=True)

        plsc.subcore_barrier()

        pltpu.sync_copy(acc_sh.at[pl.ds(base, rows_per_tile)],
                        agg_hbm.at[c, pl.ds(base, rows_per_tile)])
        pltpu.sync_copy(deg_sh.at[pl.ds(base, rows_per_tile)], degv)
        pltpu.sync_copy(degv,
                        deg_hbm.at[pl.ds(c * n_acc + base, rows_per_tile)])

    return agg_kernel(feat, src3, dst3, w3)


def _combine(feat, agg0, agg1, deg0, deg1, wn_t, ws_t):
    n, d = feat.shape
    blk = 2000
    assert n % blk == 0

    def body(feat_ref, a0_ref, a1_ref, d0_ref, d1_ref, wn_ref, ws_ref, out_ref):
        acc = a0_ref[...] + a1_ref[...]
        deg = jnp.maximum(d0_ref[...] + d1_ref[...], 1.0)
        neigh = jnp.dot(acc, wn_ref[...], preferred_element_type=jnp.float32)
        self_t = jnp.dot(feat_ref[...], ws_ref[...],
                         preferred_element_type=jnp.float32)
        out_ref[...] = self_t + neigh / deg

    return pl.pallas_call(
        body,
        grid=(n // blk,),
        in_specs=[
            pl.BlockSpec((blk, d), lambda i: (i, 0)),
            pl.BlockSpec((blk, d), lambda i: (i, 0)),
            pl.BlockSpec((blk, d), lambda i: (i, 0)),
            pl.BlockSpec((blk, 1), lambda i: (i, 0)),
            pl.BlockSpec((blk, 1), lambda i: (i, 0)),
            pl.BlockSpec((d, d), lambda i: (0, 0)),
            pl.BlockSpec((d, d), lambda i: (0, 0)),
        ],
        out_specs=pl.BlockSpec((blk, d), lambda i: (i, 0)),
        out_shape=jax.ShapeDtypeStruct((n, d), jnp.float32),
    )(feat, agg0, agg1, deg0[:, None], deg1[:, None], wn_t, ws_t)


def kernel(feat, edge_index, edge_weight, W_neigh, W_self):
    n = feat.shape[0]
    e = edge_index.shape[1]
    src = edge_index[0]
    dst = edge_index[1]

    per_round = NW * CHUNK
    nch = -(-e // per_round)
    e_pad = per_round * nch
    pad = e_pad - e
    n_acc = -(-(n + 1) // (NS * 8)) * (NS * 8)

    src_p = jnp.concatenate([src, jnp.zeros((pad,), jnp.int32)])
    dst_p = jnp.concatenate([dst, jnp.full((pad,), n, jnp.int32)])
    w_p = jnp.concatenate([edge_weight, jnp.zeros((pad,), jnp.float32)])
    src3 = src_p.reshape(NW, nch, CHUNK)
    dst3 = dst_p.reshape(NW, nch, CHUNK)
    w3 = w_p.reshape(NW, nch, CHUNK)

    agg, deg = _sc_aggregate(feat, src3, dst3, w3, n_acc, nch)
    return _combine(feat, agg[0, :n], agg[1, :n], deg[:n], deg[n_acc:n_acc + n],
                    W_neigh.T, W_self.T)
